# baseline (speedup 1.0000x reference)
import jax
import jax.numpy as jnp
from jax import lax
from jax.experimental import pallas as pl
from jax.experimental.pallas import tpu as pltpu
from jax.experimental.pallas import tpu_sc as plsc

N_MESH = 50000
N_PIVOT = 2048
D_FEAT = 256
CHUNK = 2048
N_PAD = 51200
N_CHUNKS = N_PAD // CHUNK
INF = 3.0e38


def _insert(e, j, b0, b1, b2, i0, i1, i2):
    c2 = e < b2
    c1 = e < b1
    c0 = e < b0
    nb2 = jnp.where(c2, jnp.where(c1, b1, e), b2)
    ni2 = jnp.where(c2, jnp.where(c1, i1, j), i2)
    nb1 = jnp.where(c1, jnp.where(c0, b0, e), b1)
    ni1 = jnp.where(c1, jnp.where(c0, i0, j), i1)
    nb0 = jnp.where(c0, e, b0)
    ni0 = jnp.where(c0, j, i0)
    return nb0, nb1, nb2, ni0, ni1, ni2


def _top3_body(px_ref, qt_ref, qq_ref, pp_ref, out_ref, bv_s, bi_s):
    c = pl.program_id(0)

    @pl.when(c == 0)
    def _init():
        bv_s[...] = jnp.full((3, N_PIVOT), INF, jnp.float32)
        bi_s[...] = jnp.zeros((3, N_PIVOT), jnp.float32)

    dot = lax.dot_general(
        px_ref[...], qt_ref[...], (((1,), (0,)), ((), ())),
        precision=lax.Precision.DEFAULT,
        preferred_element_type=jnp.float32)
    d2 = (qq_ref[...] - 2.0 * dot) + pp_ref[...]
    lidx = lax.broadcasted_iota(
        jnp.int32, (CHUNK, 1), 0).astype(jnp.float32)

    m1 = jnp.min(d2, axis=0, keepdims=True)
    r1 = jnp.where(d2 == m1, lidx, INF)
    a1 = jnp.min(r1, axis=0, keepdims=True)
    d2b = jnp.where(r1 == a1, INF, d2)
    m2 = jnp.min(d2b, axis=0, keepdims=True)
    r2 = jnp.where(d2b == m2, lidx, INF)
    a2 = jnp.min(r2, axis=0, keepdims=True)
    d2c = jnp.where(r2 == a2, INF, d2b)
    m3 = jnp.min(d2c, axis=0, keepdims=True)
    a3 = jnp.min(jnp.where(d2c == m3, lidx, INF), axis=0, keepdims=True)

    off = jnp.float32(c * CHUNK)
    bv = bv_s[...]
    bi = bi_s[...]
    b0, b1, b2 = bv[0:1, :], bv[1:2, :], bv[2:3, :]
    i0, i1, i2 = bi[0:1, :], bi[1:2, :], bi[2:3, :]
    b0, b1, b2, i0, i1, i2 = _insert(m1, a1 + off, b0, b1, b2, i0, i1, i2)
    b0, b1, b2, i0, i1, i2 = _insert(m2, a2 + off, b0, b1, b2, i0, i1, i2)
    b0, b1, b2, i0, i1, i2 = _insert(m3, a3 + off, b0, b1, b2, i0, i1, i2)
    bv_s[...] = jnp.concatenate([b0, b1, b2], axis=0)
    bi_s[...] = jnp.concatenate([i0, i1, i2], axis=0)

    @pl.when(c == N_CHUNKS - 1)
    def _emit():
        out_ref[...] = jnp.concatenate(
            [i0, i1, i2], axis=0).astype(jnp.int32)


def _knn_top3(q8, pxt8, qq, pp):
    return pl.pallas_call(
        _top3_body,
        grid=(N_CHUNKS,),
        in_specs=[
            pl.BlockSpec((CHUNK, 8), lambda c: (c, 0)),
            pl.BlockSpec((8, N_PIVOT), lambda c: (0, 0)),
            pl.BlockSpec((1, N_PIVOT), lambda c: (0, 0)),
            pl.BlockSpec((CHUNK, 1), lambda c: (c, 0)),
        ],
        out_specs=pl.BlockSpec((3, N_PIVOT), lambda c: (0, 0)),
        out_shape=jax.ShapeDtypeStruct((3, N_PIVOT), jnp.int32),
        scratch_shapes=[
            pltpu.VMEM((3, N_PIVOT), jnp.float32),
            pltpu.VMEM((3, N_PIVOT), jnp.float32),
        ],
        compiler_params=pltpu.CompilerParams(
            dimension_semantics=("arbitrary",)),
    )(q8, pxt8, qq, pp)


_NC = 2
_NS = 16
_NW = _NC * _NS
_QPW = N_PIVOT // _NW
_RPW = 3 * _QPW


def _interp_body(x_hbm, px_hbm, py_hbm, xidx_hbm, y_hbm, w_hbm,
                 idx_v, xrows, prows, qrows, ybuf, wbuf, sem):
    wid = lax.axis_index("s") * _NC + lax.axis_index("c")
    qbase = wid * _QPW
    rbase = wid * _RPW
    pltpu.sync_copy(xidx_hbm.at[pl.ds(rbase, _RPW)], idx_v)
    pltpu.async_copy(x_hbm.at[idx_v], xrows, sem).wait()
    pltpu.async_copy(px_hbm.at[idx_v], prows, sem).wait()
    pltpu.sync_copy(py_hbm.at[pl.ds(qbase, _QPW)], qrows)

    lane = lax.iota(jnp.int32, 16)

    def splat(v, i):
        idx = jnp.full((16, 1), i, dtype=jnp.int32)
        dn = lax.GatherDimensionNumbers(
            offset_dims=(), collapsed_slice_dims=(0,), start_index_map=(0,))
        return lax.gather(v, idx, dn, slice_sizes=(1,),
                          mode=lax.GatherScatterMode.PROMISE_IN_BOUNDS)

    def body(q, _):
        qv = qrows[q, :]
        ws = []
        for s in range(3):
            pv = prows[3 * q + s, pl.ds(0, 16)]
            dv = pv - qv
            sq = dv * dv
            d2v = (splat(sq, 0) + splat(sq, 1)) + splat(sq, 2)
            wv = 1.0 / jnp.maximum(d2v, jnp.float32(1e-16))
            ws.append(wv)
        w0, w1, w2 = ws
        wvec = jnp.where(lane == 0, w0,
                         jnp.where(lane == 1, w1,
                                   jnp.where(lane == 2, w2, 0.0)))
        wbuf[q, :] = wvec
        inv = 1.0 / ((w0 + w1) + w2)
        for f in range(D_FEAT // 16):
            sl = pl.ds(f * 16, 16)
            acc = (xrows[3 * q, sl] * w0 + xrows[3 * q + 1, sl] * w1
                   + xrows[3 * q + 2, sl] * w2)
            ybuf[q, sl] = acc * inv
        return _

    lax.fori_loop(0, _QPW, body, 0)
    pltpu.sync_copy(ybuf, y_hbm.at[pl.ds(qbase, _QPW)])
    pltpu.sync_copy(wbuf, w_hbm.at[pl.ds(qbase, _QPW)])


def _interp(x, px_pad, py_pad, xidx_flat):
    mesh = plsc.VectorSubcoreMesh(core_axis_name="c", subcore_axis_name="s")
    fn = pl.kernel(
        _interp_body,
        mesh=mesh,
        out_type=[
            jax.ShapeDtypeStruct((N_PIVOT, D_FEAT), jnp.float32),
            jax.ShapeDtypeStruct((N_PIVOT, 16), jnp.float32),
        ],
        scratch_types=[
            pltpu.VMEM((_RPW,), jnp.int32),
            pltpu.VMEM((_RPW, D_FEAT), jnp.float32),
            pltpu.VMEM((_RPW, 128), jnp.float32),
            pltpu.VMEM((_QPW, 16), jnp.float32),
            pltpu.VMEM((_QPW, D_FEAT), jnp.float32),
            pltpu.VMEM((_QPW, 16), jnp.float32),
            pltpu.SemaphoreType.DMA,
        ],
    )
    return fn(x, px_pad, py_pad, xidx_flat)


def kernel(x, pos_x, pos_y, k):
    f32 = jnp.float32
    x_sq = jnp.sum(pos_x * pos_x, axis=-1)
    qq = jnp.sum(pos_y * pos_y, axis=-1, keepdims=True)
    px8 = jnp.concatenate(
        [pos_x, jnp.zeros((N_MESH, 5), f32)], axis=1)
    px8 = jnp.concatenate([px8, jnp.zeros((N_PAD - N_MESH, 8), f32)], axis=0)
    q8t = jnp.concatenate([pos_y, jnp.zeros((N_PIVOT, 5), f32)], axis=1).T
    pp = jnp.concatenate(
        [x_sq, jnp.full((N_PAD - N_MESH,), 3.0e38, f32)])[:, None]

    bi = _knn_top3(px8, q8t, qq.T, pp)
    x_idx = bi.T.reshape(-1)

    px_pad = jnp.concatenate([pos_x, jnp.zeros((N_MESH, 125), f32)], axis=1)
    py_pad = jnp.concatenate([pos_y, jnp.zeros((N_PIVOT, 13), f32)], axis=1)
    y, w = _interp(x, px_pad, py_pad, x_idx)
    weights = w[:, :3].reshape(-1)[:, None]

    y_idx = jnp.repeat(jnp.arange(N_PIVOT, dtype=jnp.int32), 3)
    return (y, x_idx, y_idx, weights)
